# baseline (speedup 1.0000x reference)
import jax
import jax.numpy as jnp
from jax import lax
from jax.experimental import pallas as pl
from jax.experimental.pallas import tpu as pltpu
from jax.experimental.pallas import tpu_sc as plsc

B = 16384
EMB = 64
N = 100000
NW = 32
NBLK = 782
HITCAP = 768
NP = (9, 8, 8)
DUMP = B
STAGE_ROWS = B + 16


def _gather_body(uid_hbm, pid_hbm, ut_hbm, pt_hbm, tu_hbm, tp_hbm,
                 su_hbm, sp_hbm,
                 idsu, idsp, hbu, hbp, pb, pu, blockbuf, sstage, t1, sidx,
                 sem_ids, sem_blk, sem_sc):
    w = lax.axis_index("s") * 2 + lax.axis_index("c")
    lanes = lax.iota(jnp.int32, 16)

    cu_ids = pltpu.async_copy(uid_hbm.at[pl.ds(0, B)], idsu, sem_ids)
    cp_ids = pltpu.async_copy(pid_hbm.at[pl.ds(0, B)], idsp, sem_ids)
    cu_ids.wait()
    cp_ids.wait()

    def binit(i, carry):
        cu, cp = carry
        u16 = idsu[pl.ds(i * 16, 16)]
        p16 = idsp[pl.ds(i * 16, 16)]
        b16 = i * 16 + lanes
        mu = ((u16 >> 7) & 31) == w
        mp = ((p16 >> 7) & 31) == w
        plsc.store_compressed(hbu.at[pl.ds(cu, 16)], b16, mask=mu)
        plsc.store_compressed(hbp.at[pl.ds(cp, 16)], b16, mask=mp)
        cu = jnp.minimum(cu + plsc.all_reduce_population_count(mu)[0], HITCAP)
        cp = jnp.minimum(cp + plsc.all_reduce_population_count(mp)[0], HITCAP)
        return cu, cp

    hcnt_u, hcnt_p = lax.fori_loop(0, B // 16, binit, (0, 0))

    kidx = [lax.iota(jnp.int32, 16) + 16 * c for c in range(4)]

    for tab_hbm, tail_hbm, ids, hb, hcnt, stage_hbm in (
            (ut_hbm, tu_hbm, idsu, hbu, hcnt_u, su_hbm),
            (pt_hbm, tp_hbm, idsp, hbp, hcnt_p, sp_hbm)):
        for p in range(len(NP)):
            jlo = sum(NP[:p])
            copies = []
            for jj in range(NP[p]):
                blk = jnp.minimum(w + 32 * (jlo + jj), NBLK - 2)
                copies.append(pltpu.async_copy(
                    tab_hbm.at[:, pl.ds(blk * 128, 128)],
                    blockbuf.at[:, pl.ds(jj * 128, 128)], sem_blk))
            if p == len(NP) - 1:
                copies.append(pltpu.async_copy(
                    tail_hbm.at[:, pl.ds(0, 128)],
                    blockbuf.at[:, pl.ds(1152, 128)], sem_blk))

            def passit(i, pcnt, hb=hb, ids=ids, hcnt=hcnt, jlo=jlo, np_=NP[p]):
                b16 = hb[pl.ds(i * 16, 16)] & (B - 1)
                u16 = plsc.load_gather(ids, [b16])
                j16 = u16 >> 12
                m = ((i * 16 + lanes) < hcnt) & (j16 >= jlo) & (j16 < jlo + np_)
                plsc.store_compressed(pb.at[pl.ds(pcnt, 16)], b16, mask=m)
                plsc.store_compressed(pu.at[pl.ds(pcnt, 16)], u16, mask=m)
                return jnp.minimum(pcnt + jnp.sum(m.astype(jnp.int32)), HITCAP)

            pcnt = lax.fori_loop(0, HITCAP // 16 + 1, passit, 0)

            for c in copies:
                c.wait()

            nch = (pcnt + 15) >> 4

            def chunk(i, carry, jlo=jlo, stage_hbm=stage_hbm):
                b16 = pb[pl.ds(i * 16, 16)]
                u16 = pu[pl.ds(i * 16, 16)]
                posm = (i * 16 + lanes) < pcnt
                jj16 = (u16 >> 12) - jlo
                col = jj16 * 128 + (u16 & 127)
                col = jnp.where((u16 >> 7) == (NBLK - 1),
                                1152 + 96 + (u16 & 127), col)
                col = jnp.where(posm, col, 0)
                s = i & 3

                @pl.when(i >= 4)
                def _():
                    pltpu.make_async_copy(
                        stage_hbm.at[pl.ds(DUMP, 16)], sstage.at[s],
                        sem_sc).wait()

                sidx[s, pl.ds(0, 16)] = jnp.where(posm, b16, DUMP)
                rowbase = lanes * 129
                for k in range(EMB):
                    g = plsc.load_gather(blockbuf,
                                         [jnp.full((16,), k, jnp.int32), col])
                    plsc.store_scatter(t1, [rowbase + k], g)
                for l in range(16):
                    for c in range(4):
                        sstage[s, l, pl.ds(c * 16, 16)] = (
                            t1[pl.ds(l * 129 + c * 16, 16)])
                pltpu.async_copy(sstage.at[s], stage_hbm.at[sidx.at[s]],
                                 sem_sc)
                return carry

            lax.fori_loop(0, nch, chunk, 0)

            def drain(i, carry, stage_hbm=stage_hbm):
                pltpu.make_async_copy(stage_hbm.at[pl.ds(DUMP, 16)],
                                      sstage.at[i & 3], sem_sc).wait()
                return carry

            lax.fori_loop(0, jnp.minimum(nch, 4), drain, 0)


def _dot_tc(u_ref, p_ref, o_ref):
    o_ref[...] = jnp.sum(u_ref[:, :EMB] * p_ref[:, :EMB], axis=1,
                         keepdims=True)


@jax.jit
def kernel(user_ids, place_ids, user_table, place_table):
    mesh = plsc.VectorSubcoreMesh(core_axis_name="c", subcore_axis_name="s")
    k1 = pl.kernel(
        _gather_body,
        out_type=(jax.ShapeDtypeStruct((STAGE_ROWS, 128), jnp.float32),
                  jax.ShapeDtypeStruct((STAGE_ROWS, 128), jnp.float32)),
        mesh=mesh,
        scratch_types=[
            pltpu.VMEM((B,), jnp.int32),
            pltpu.VMEM((B,), jnp.int32),
            pltpu.VMEM((HITCAP + 16,), jnp.int32),
            pltpu.VMEM((HITCAP + 16,), jnp.int32),
            pltpu.VMEM((HITCAP + 16,), jnp.int32),
            pltpu.VMEM((HITCAP + 16,), jnp.int32),
            pltpu.VMEM((64, 1280), jnp.float32),
            pltpu.VMEM((4, 16, 128), jnp.float32),
            pltpu.VMEM((16 * 129 + 16,), jnp.float32),
            pltpu.VMEM((4, 16), jnp.int32),
            pltpu.SemaphoreType.DMA,
            pltpu.SemaphoreType.DMA,
            pltpu.SemaphoreType.DMA,
        ],
        compiler_params=pltpu.CompilerParams(
            needs_layout_passes=False, use_tc_tiling_on_sc=True),
    )
    ut = user_table.T
    pt = place_table.T
    stage_u, stage_p = k1(user_ids.astype(jnp.int32),
                          place_ids.astype(jnp.int32),
                          ut, pt, ut[:, N - 128:], pt[:, N - 128:])

    out = pl.pallas_call(
        _dot_tc,
        grid=(NW,),
        in_specs=[pl.BlockSpec((B // NW, 128), lambda i: (i, 0)),
                  pl.BlockSpec((B // NW, 128), lambda i: (i, 0))],
        out_specs=pl.BlockSpec((B // NW, 1), lambda i: (i, 0)),
        out_shape=jax.ShapeDtypeStruct((B, 1), jnp.float32),
    )(stage_u, stage_p)
    return out


# speedup vs baseline: 1.0892x; 1.0892x over previous
import jax
import jax.numpy as jnp
from jax import lax
from jax.experimental import pallas as pl
from jax.experimental.pallas import tpu as pltpu
from jax.experimental.pallas import tpu_sc as plsc

B = 16384
EMB = 64
N = 100000
NW = 32
NBLK = 782
HITCAP = 768
NP = (9, 8, 8)
DUMP = B
STAGE_ROWS = B + 16


def _gather_body(uid_hbm, pid_hbm, ut_hbm, pt_hbm, tu_hbm, tp_hbm,
                 su_hbm, sp_hbm,
                 idsu, idsp, hbu, hbp, pb, pu, blockbuf, sstage, t1, sidx,
                 sem_ids, sem_blk, sem_sc):
    w = lax.axis_index("s") * 2 + lax.axis_index("c")
    lanes = lax.iota(jnp.int32, 16)

    cu_ids = pltpu.async_copy(uid_hbm.at[pl.ds(0, B)], idsu, sem_ids)
    cp_ids = pltpu.async_copy(pid_hbm.at[pl.ds(0, B)], idsp, sem_ids)
    cu_ids.wait()
    cp_ids.wait()

    def binit(i, carry):
        cu, cp = carry
        u16 = idsu[pl.ds(i * 16, 16)]
        p16 = idsp[pl.ds(i * 16, 16)]
        b16 = i * 16 + lanes
        mu = ((u16 >> 7) & 31) == w
        mp = ((p16 >> 7) & 31) == w
        plsc.store_compressed(hbu.at[pl.ds(cu, 16)], b16, mask=mu)
        plsc.store_compressed(hbp.at[pl.ds(cp, 16)], b16, mask=mp)
        cu = jnp.minimum(cu + plsc.all_reduce_population_count(mu)[0], HITCAP)
        cp = jnp.minimum(cp + plsc.all_reduce_population_count(mp)[0], HITCAP)
        return cu, cp

    hcnt_u, hcnt_p = lax.fori_loop(0, B // 16, binit, (0, 0))

    kidx = [lax.iota(jnp.int32, 16) + 16 * c for c in range(4)]

    ci = 0
    for tab_hbm, tail_hbm, ids, hb, hcnt, stage_hbm in (
            (ut_hbm, tu_hbm, idsu, hbu, hcnt_u, su_hbm),
            (pt_hbm, tp_hbm, idsp, hbp, hcnt_p, sp_hbm)):
        for p in range(len(NP)):
            jlo = sum(NP[:p])
            copies = []
            for jj in range(NP[p]):
                blk = jnp.minimum(w + 32 * (jlo + jj), NBLK - 2)
                copies.append(pltpu.async_copy(
                    tab_hbm.at[:, pl.ds(blk * 128, 128)],
                    blockbuf.at[:, pl.ds(jj * 128, 128)], sem_blk))
            if p == len(NP) - 1:
                copies.append(pltpu.async_copy(
                    tail_hbm.at[:, pl.ds(0, 128)],
                    blockbuf.at[:, pl.ds(1152, 128)], sem_blk))

            def passit(i, pcnt, hb=hb, ids=ids, hcnt=hcnt, jlo=jlo, np_=NP[p]):
                b16 = hb[pl.ds(i * 16, 16)] & (B - 1)
                u16 = plsc.load_gather(ids, [b16])
                j16 = u16 >> 12
                m = ((i * 16 + lanes) < hcnt) & (j16 >= jlo) & (j16 < jlo + np_)
                plsc.store_compressed(pb.at[pl.ds(pcnt, 16)], b16, mask=m)
                plsc.store_compressed(pu.at[pl.ds(pcnt, 16)], u16, mask=m)
                return jnp.minimum(pcnt + jnp.sum(m.astype(jnp.int32)), HITCAP)

            pcnt = lax.fori_loop(0, HITCAP // 16 + 1, passit, 0)

            for c in copies:
                c.wait()

            nch = (pcnt + 15) >> 4

            def chunk(i, ci, jlo=jlo, stage_hbm=stage_hbm):
                b16 = pb[pl.ds(i * 16, 16)]
                u16 = pu[pl.ds(i * 16, 16)]
                posm = (i * 16 + lanes) < pcnt
                jj16 = (u16 >> 12) - jlo
                col = jj16 * 128 + (u16 & 127)
                col = jnp.where((u16 >> 7) == (NBLK - 1),
                                1152 + 96 + (u16 & 127), col)
                col = jnp.where(posm, col, 0)
                s = ci & 3

                @pl.when(ci >= 4)
                def _():
                    pltpu.make_async_copy(
                        stage_hbm.at[pl.ds(DUMP, 16)], sstage.at[s],
                        sem_sc).wait()

                sidx[s, pl.ds(0, 16)] = jnp.where(posm, b16, DUMP)
                rowbase = lanes * 129
                for k in range(EMB):
                    g = plsc.load_gather(blockbuf,
                                         [jnp.full((16,), k, jnp.int32), col])
                    plsc.store_scatter(t1, [rowbase + k], g)
                for l in range(16):
                    for c in range(4):
                        sstage[s, l, pl.ds(c * 16, 16)] = (
                            t1[pl.ds(l * 129 + c * 16, 16)])
                pltpu.async_copy(sstage.at[s], stage_hbm.at[sidx.at[s]],
                                 sem_sc)
                return ci + 1

            ci = lax.fori_loop(0, nch, chunk, ci)

    def drain(i, carry):
        pltpu.make_async_copy(su_hbm.at[pl.ds(DUMP, 16)],
                              sstage.at[i & 3], sem_sc).wait()
        return carry

    lax.fori_loop(0, jnp.minimum(ci, 4), drain, 0)


def _dot_tc(u_ref, p_ref, o_ref):
    o_ref[...] = jnp.sum(u_ref[:, :EMB] * p_ref[:, :EMB], axis=1,
                         keepdims=True)


@jax.jit
def kernel(user_ids, place_ids, user_table, place_table):
    mesh = plsc.VectorSubcoreMesh(core_axis_name="c", subcore_axis_name="s")
    k1 = pl.kernel(
        _gather_body,
        out_type=(jax.ShapeDtypeStruct((STAGE_ROWS, 128), jnp.float32),
                  jax.ShapeDtypeStruct((STAGE_ROWS, 128), jnp.float32)),
        mesh=mesh,
        scratch_types=[
            pltpu.VMEM((B,), jnp.int32),
            pltpu.VMEM((B,), jnp.int32),
            pltpu.VMEM((HITCAP + 16,), jnp.int32),
            pltpu.VMEM((HITCAP + 16,), jnp.int32),
            pltpu.VMEM((HITCAP + 16,), jnp.int32),
            pltpu.VMEM((HITCAP + 16,), jnp.int32),
            pltpu.VMEM((64, 1280), jnp.float32),
            pltpu.VMEM((4, 16, 128), jnp.float32),
            pltpu.VMEM((16 * 129 + 16,), jnp.float32),
            pltpu.VMEM((4, 16), jnp.int32),
            pltpu.SemaphoreType.DMA,
            pltpu.SemaphoreType.DMA,
            pltpu.SemaphoreType.DMA,
        ],
        compiler_params=pltpu.CompilerParams(
            needs_layout_passes=False, use_tc_tiling_on_sc=True),
    )
    ut = user_table.T
    pt = place_table.T
    stage_u, stage_p = k1(user_ids.astype(jnp.int32),
                          place_ids.astype(jnp.int32),
                          ut, pt, ut[:, N - 128:], pt[:, N - 128:])

    out = pl.pallas_call(
        _dot_tc,
        grid=(8,),
        in_specs=[pl.BlockSpec((B // 8, 128), lambda i: (i, 0)),
                  pl.BlockSpec((B // 8, 128), lambda i: (i, 0))],
        out_specs=pl.BlockSpec((B // 8, 1), lambda i: (i, 0)),
        out_shape=jax.ShapeDtypeStruct((B, 1), jnp.float32),
    )(stage_u, stage_p)
    return out


# speedup vs baseline: 1.1009x; 1.0108x over previous
import jax
import jax.numpy as jnp
from jax import lax
from jax.experimental import pallas as pl
from jax.experimental.pallas import tpu as pltpu
from jax.experimental.pallas import tpu_sc as plsc

B = 16384
EMB = 64
N = 100000
NW = 32
NBLK = 782
HITCAP = 768
NP = (9, 8, 8)
DUMP = B
STAGE_ROWS = B + 16


def _gather_body(uid_hbm, pid_hbm, ut_hbm, pt_hbm, tu_hbm, tp_hbm,
                 su_hbm, sp_hbm,
                 idsu, idsp, hbu, hbp, pb, pu, blockbuf, sstage, t1, sidx,
                 sem_ids, sem_blk, sem_sc):
    w = lax.axis_index("s") * 2 + lax.axis_index("c")
    lanes = lax.iota(jnp.int32, 16)

    cu_ids = pltpu.async_copy(uid_hbm.at[pl.ds(0, B)], idsu, sem_ids)
    cp_ids = pltpu.async_copy(pid_hbm.at[pl.ds(0, B)], idsp, sem_ids)
    cu_ids.wait()
    cp_ids.wait()

    def binit(i, carry):
        cu, cp = carry
        u16 = idsu[pl.ds(i * 16, 16)]
        p16 = idsp[pl.ds(i * 16, 16)]
        b16 = i * 16 + lanes
        mu = ((u16 >> 7) & 31) == w
        mp = ((p16 >> 7) & 31) == w
        plsc.store_compressed(hbu.at[pl.ds(cu, 16)], b16, mask=mu)
        plsc.store_compressed(hbp.at[pl.ds(cp, 16)], b16, mask=mp)
        cu = jnp.minimum(cu + plsc.all_reduce_population_count(mu)[0], HITCAP)
        cp = jnp.minimum(cp + plsc.all_reduce_population_count(mp)[0], HITCAP)
        return cu, cp

    hcnt_u, hcnt_p = lax.fori_loop(0, B // 16, binit, (0, 0))

    kidx = [lax.iota(jnp.int32, 16) + 16 * c for c in range(4)]

    ci = 0
    for tab_hbm, tail_hbm, ids, hb, hcnt, stage_hbm in (
            (ut_hbm, tu_hbm, idsu, hbu, hcnt_u, su_hbm),
            (pt_hbm, tp_hbm, idsp, hbp, hcnt_p, sp_hbm)):
        for p in range(len(NP)):
            jlo = sum(NP[:p])
            copies = []
            for jj in range(NP[p]):
                blk = jnp.minimum(w + 32 * (jlo + jj), NBLK - 2)
                copies.append(pltpu.async_copy(
                    tab_hbm.at[:, pl.ds(blk * 128, 128)],
                    blockbuf.at[:, pl.ds(jj * 128, 128)], sem_blk))
            if p == len(NP) - 1:
                copies.append(pltpu.async_copy(
                    tail_hbm.at[:, pl.ds(0, 128)],
                    blockbuf.at[:, pl.ds(1152, 128)], sem_blk))

            def passit(i, pcnt, hb=hb, ids=ids, hcnt=hcnt, jlo=jlo, np_=NP[p]):
                b16 = hb[pl.ds(i * 16, 16)] & (B - 1)
                u16 = plsc.load_gather(ids, [b16])
                j16 = u16 >> 12
                m = ((i * 16 + lanes) < hcnt) & (j16 >= jlo) & (j16 < jlo + np_)
                plsc.store_compressed(pb.at[pl.ds(pcnt, 16)], b16, mask=m)
                plsc.store_compressed(pu.at[pl.ds(pcnt, 16)], u16, mask=m)
                return jnp.minimum(pcnt + jnp.sum(m.astype(jnp.int32)), HITCAP)

            pcnt = lax.fori_loop(0, HITCAP // 16 + 1, passit, 0)

            for c in copies:
                c.wait()

            nch = (pcnt + 15) >> 4

            def chunk(i, ci, jlo=jlo, stage_hbm=stage_hbm):
                b16 = pb[pl.ds(i * 16, 16)]
                u16 = pu[pl.ds(i * 16, 16)]
                posm = (i * 16 + lanes) < pcnt
                jj16 = (u16 >> 12) - jlo
                col = jj16 * 128 + (u16 & 127)
                col = jnp.where((u16 >> 7) == (NBLK - 1),
                                1152 + 96 + (u16 & 127), col)
                col = jnp.where(posm, col, 0)
                s = ci & 3

                @pl.when(ci >= 4)
                def _():
                    pltpu.make_async_copy(
                        stage_hbm.at[pl.ds(DUMP, 16)], sstage.at[s],
                        sem_sc).wait()

                sidx[s, pl.ds(0, 16)] = jnp.where(posm, b16, DUMP)
                rowbase = lanes * 129
                for k in range(EMB):
                    g = plsc.load_gather(blockbuf,
                                         [jnp.full((16,), k, jnp.int32), col])
                    plsc.store_scatter(t1, [rowbase + k], g)
                for l in range(16):
                    for c in range(4):
                        sstage[s, l, pl.ds(c * 16, 16)] = (
                            t1[pl.ds(l * 129 + c * 16, 16)])
                pltpu.async_copy(sstage.at[s], stage_hbm.at[sidx.at[s]],
                                 sem_sc)
                return ci + 1

            ci = lax.fori_loop(0, nch, chunk, ci)

    def drain(i, carry):
        pltpu.make_async_copy(su_hbm.at[pl.ds(DUMP, 16)],
                              sstage.at[i & 3], sem_sc).wait()
        return carry

    lax.fori_loop(0, jnp.minimum(ci, 4), drain, 0)


def _dot_tc(u_ref, p_ref, o_ref):
    o_ref[...] = jnp.sum(u_ref[:, :EMB] * p_ref[:, :EMB], axis=1,
                         keepdims=True)


@jax.jit
def kernel(user_ids, place_ids, user_table, place_table):
    mesh = plsc.VectorSubcoreMesh(core_axis_name="c", subcore_axis_name="s")
    k1 = pl.kernel(
        _gather_body,
        out_type=(jax.ShapeDtypeStruct((STAGE_ROWS, 128), jnp.float32),
                  jax.ShapeDtypeStruct((STAGE_ROWS, 128), jnp.float32)),
        mesh=mesh,
        scratch_types=[
            pltpu.VMEM((B,), jnp.int32),
            pltpu.VMEM((B,), jnp.int32),
            pltpu.VMEM((HITCAP + 16,), jnp.int32),
            pltpu.VMEM((HITCAP + 16,), jnp.int32),
            pltpu.VMEM((HITCAP + 16,), jnp.int32),
            pltpu.VMEM((HITCAP + 16,), jnp.int32),
            pltpu.VMEM((64, 1280), jnp.float32),
            pltpu.VMEM((4, 16, 128), jnp.float32),
            pltpu.VMEM((16 * 129 + 16,), jnp.float32),
            pltpu.VMEM((4, 16), jnp.int32),
            pltpu.SemaphoreType.DMA,
            pltpu.SemaphoreType.DMA,
            pltpu.SemaphoreType.DMA,
        ],
        compiler_params=pltpu.CompilerParams(
            needs_layout_passes=False, use_tc_tiling_on_sc=True),
    )
    ut = user_table.T
    pt = place_table.T
    stage_u, stage_p = k1(user_ids.astype(jnp.int32),
                          place_ids.astype(jnp.int32),
                          ut, pt, ut[:, N - 128:], pt[:, N - 128:])

    out = pl.pallas_call(
        _dot_tc,
        grid=(4,),
        in_specs=[pl.BlockSpec((B // 4, 128), lambda i: (i, 0)),
                  pl.BlockSpec((B // 4, 128), lambda i: (i, 0))],
        out_specs=pl.BlockSpec((B // 4, 1), lambda i: (i, 0)),
        out_shape=jax.ShapeDtypeStruct((B, 1), jnp.float32),
    )(stage_u, stage_p)
    return out


# speedup vs baseline: 1.1436x; 1.0387x over previous
import jax
import jax.numpy as jnp
from jax import lax
from jax.experimental import pallas as pl
from jax.experimental.pallas import tpu as pltpu
from jax.experimental.pallas import tpu_sc as plsc

B = 16384
EMB = 64
N = 100000
NW = 32
NBLK = 782
HITCAP = 768
NP = (9, 8, 8)
DUMP = B
STAGE_ROWS = B + 16


def _gather_body(uid_hbm, pid_hbm, ut_hbm, pt_hbm, tu_hbm, tp_hbm,
                 su_hbm, sp_hbm,
                 idsu, idsp, hbu, hbp, pb, pu, blockbuf, sstage, t1, sidx,
                 sem_ids, sem_blk, sem_sc):
    w = lax.axis_index("s") * 2 + lax.axis_index("c")
    lanes = lax.iota(jnp.int32, 16)

    cu_ids = pltpu.async_copy(uid_hbm.at[pl.ds(0, B)], idsu, sem_ids)
    cp_ids = pltpu.async_copy(pid_hbm.at[pl.ds(0, B)], idsp, sem_ids)
    cu_ids.wait()
    cp_ids.wait()

    def binit(i, carry):
        cu, cp = carry
        u16 = idsu[pl.ds(i * 16, 16)]
        p16 = idsp[pl.ds(i * 16, 16)]
        b16 = i * 16 + lanes
        mu = ((u16 >> 7) & 31) == w
        mp = ((p16 >> 7) & 31) == w
        plsc.store_compressed(hbu.at[pl.ds(cu, 16)], b16, mask=mu)
        plsc.store_compressed(hbp.at[pl.ds(cp, 16)], b16, mask=mp)
        cu = jnp.minimum(cu + plsc.all_reduce_population_count(mu)[0], HITCAP)
        cp = jnp.minimum(cp + plsc.all_reduce_population_count(mp)[0], HITCAP)
        return cu, cp

    hcnt_u, hcnt_p = lax.fori_loop(0, B // 16, binit, (0, 0))

    kidx = [lax.iota(jnp.int32, 16) + 16 * c for c in range(4)]

    ci = 0
    for tab_hbm, tail_hbm, ids, hb, hcnt, stage_hbm in (
            (ut_hbm, tu_hbm, idsu, hbu, hcnt_u, su_hbm),
            (pt_hbm, tp_hbm, idsp, hbp, hcnt_p, sp_hbm)):
        for p in range(len(NP)):
            jlo = sum(NP[:p])
            copies = []
            for jj in range(NP[p]):
                blk = jnp.minimum(w + 32 * (jlo + jj), NBLK - 2)
                copies.append(pltpu.async_copy(
                    tab_hbm.at[:, pl.ds(blk * 128, 128)],
                    blockbuf.at[:, pl.ds(jj * 128, 128)], sem_blk))
            if p == len(NP) - 1:
                copies.append(pltpu.async_copy(
                    tail_hbm.at[:, pl.ds(0, 128)],
                    blockbuf.at[:, pl.ds(1152, 128)], sem_blk))

            def passit(i, pcnt, hb=hb, ids=ids, hcnt=hcnt, jlo=jlo, np_=NP[p]):
                b16 = hb[pl.ds(i * 16, 16)] & (B - 1)
                u16 = plsc.load_gather(ids, [b16])
                j16 = u16 >> 12
                m = ((i * 16 + lanes) < hcnt) & (j16 >= jlo) & (j16 < jlo + np_)
                plsc.store_compressed(pb.at[pl.ds(pcnt, 16)], b16, mask=m)
                plsc.store_compressed(pu.at[pl.ds(pcnt, 16)], u16, mask=m)
                return jnp.minimum(pcnt + jnp.sum(m.astype(jnp.int32)), HITCAP)

            pcnt = lax.fori_loop(0, HITCAP // 16 + 1, passit, 0)

            for c in copies:
                c.wait()

            nch = (pcnt + 15) >> 4

            def chunk(i, ci, jlo=jlo, stage_hbm=stage_hbm):
                b16 = pb[pl.ds(i * 16, 16)]
                u16 = pu[pl.ds(i * 16, 16)]
                posm = (i * 16 + lanes) < pcnt
                jj16 = (u16 >> 12) - jlo
                col = jj16 * 128 + (u16 & 127)
                col = jnp.where((u16 >> 7) == (NBLK - 1),
                                1152 + 96 + (u16 & 127), col)
                col = jnp.where(posm, col, 0)
                s = ci & 3

                @pl.when(ci >= 4)
                def _():
                    pltpu.make_async_copy(
                        stage_hbm.at[pl.ds(DUMP, 16)], sstage.at[s],
                        sem_sc).wait()

                sidx[s, pl.ds(0, 16)] = jnp.where(posm, b16, DUMP)
                rowbase = lanes * 129
                for k in range(EMB):
                    g = plsc.load_gather(blockbuf,
                                         [jnp.full((16,), k, jnp.int32), col])
                    plsc.store_scatter(t1, [rowbase + k], g)
                for l in range(16):
                    for c in range(4):
                        sstage[s, l, pl.ds(c * 16, 16)] = (
                            t1[pl.ds(l * 129 + c * 16, 16)])
                pltpu.async_copy(sstage.at[s], stage_hbm.at[sidx.at[s]],
                                 sem_sc)
                return ci + 1

            ci = lax.fori_loop(0, nch, chunk, ci)

    def drain(i, carry):
        pltpu.make_async_copy(su_hbm.at[pl.ds(DUMP, 16)],
                              sstage.at[i & 3], sem_sc).wait()
        return carry

    lax.fori_loop(0, jnp.minimum(ci, 4), drain, 0)


def _dot_tc(u_ref, p_ref, o_ref):
    o_ref[...] = jnp.sum(u_ref[:, :EMB] * p_ref[:, :EMB], axis=1)


@jax.jit
def kernel(user_ids, place_ids, user_table, place_table):
    mesh = plsc.VectorSubcoreMesh(core_axis_name="c", subcore_axis_name="s")
    k1 = pl.kernel(
        _gather_body,
        out_type=(jax.ShapeDtypeStruct((STAGE_ROWS, 128), jnp.float32),
                  jax.ShapeDtypeStruct((STAGE_ROWS, 128), jnp.float32)),
        mesh=mesh,
        scratch_types=[
            pltpu.VMEM((B,), jnp.int32),
            pltpu.VMEM((B,), jnp.int32),
            pltpu.VMEM((HITCAP + 16,), jnp.int32),
            pltpu.VMEM((HITCAP + 16,), jnp.int32),
            pltpu.VMEM((HITCAP + 16,), jnp.int32),
            pltpu.VMEM((HITCAP + 16,), jnp.int32),
            pltpu.VMEM((64, 1280), jnp.float32),
            pltpu.VMEM((4, 16, 128), jnp.float32),
            pltpu.VMEM((16 * 129 + 16,), jnp.float32),
            pltpu.VMEM((4, 16), jnp.int32),
            pltpu.SemaphoreType.DMA,
            pltpu.SemaphoreType.DMA,
            pltpu.SemaphoreType.DMA,
        ],
        compiler_params=pltpu.CompilerParams(
            needs_layout_passes=False, use_tc_tiling_on_sc=True),
    )
    ut = user_table.T
    pt = place_table.T
    stage_u, stage_p = k1(user_ids.astype(jnp.int32),
                          place_ids.astype(jnp.int32),
                          ut, pt, ut[:, N - 128:], pt[:, N - 128:])

    out = pl.pallas_call(
        _dot_tc,
        grid=(4,),
        in_specs=[pl.BlockSpec((B // 4, 128), lambda i: (i, 0)),
                  pl.BlockSpec((B // 4, 128), lambda i: (i, 0))],
        out_specs=pl.BlockSpec((B // 4,), lambda i: (i,)),
        out_shape=jax.ShapeDtypeStruct((B,), jnp.float32),
    )(stage_u, stage_p)
    return out.reshape(B, 1)
